# baseline (speedup 1.0000x reference)
import functools

import jax
import jax.numpy as jnp
import numpy as np
from jax import lax
from jax.experimental import pallas as pl
from jax.experimental.pallas import tpu as pltpu
from jax.experimental.pallas import tpu_sc as plsc

N_NODES = 10000
N_EDGES = 320000
NODE_DIM = 128
EDGE_DIM = 16
OUT_DIM = 128
HALF = OUT_DIM // 2

_NC = 2
_NS = 16
_NW = _NC * _NS
_PER_W = N_EDGES // _NW
_C = 80
_NCHUNK = _PER_W // _C
_NRING = 4

_HI = np.int32(-65536)


def _tables_body(x_ref, w_ref, t1_ref, t2_ref):
    xv = x_ref[...]
    t1_ref[...] = jnp.dot(xv, w_ref[0:NODE_DIM, :],
                          preferred_element_type=jnp.float32)
    t2_ref[...] = jnp.dot(xv, w_ref[NODE_DIM:2 * NODE_DIM, :],
                          preferred_element_type=jnp.float32)


def _make_tables(x, W):
    return pl.pallas_call(
        _tables_body,
        out_shape=(
            jax.ShapeDtypeStruct((N_NODES, OUT_DIM), jnp.float32),
            jax.ShapeDtypeStruct((N_NODES, OUT_DIM), jnp.float32),
        ),
    )(x, W)


def _pack_words(t):
    lo = t[:, :HALF].astype(jnp.bfloat16)
    hi = t[:, HALF:].astype(jnp.bfloat16)
    return jax.lax.bitcast_convert_type(jnp.stack([lo, hi], axis=-1),
                                        jnp.int32)


_sc_mesh = plsc.VectorSubcoreMesh(core_axis_name="c", subcore_axis_name="s")


@functools.partial(
    pl.kernel,
    out_type=(
        jax.ShapeDtypeStruct((N_EDGES, HALF), jnp.int32),
        jax.ShapeDtypeStruct((N_EDGES, HALF), jnp.int32),
    ),
    mesh=_sc_mesh,
    compiler_params=pltpu.CompilerParams(use_tc_tiling_on_sc=False),
    scratch_types=[
        pltpu.VMEM((_PER_W,), jnp.int32),
        pltpu.VMEM((_PER_W,), jnp.int32),
        pltpu.VMEM((_NRING, _C, HALF), jnp.int32),
        pltpu.VMEM((_NRING, _C, HALF), jnp.int32),
        [pltpu.SemaphoreType.DMA] * _NRING,
        [pltpu.SemaphoreType.DMA] * _NRING,
    ],
)
def _sc_gather(t1_hbm, t2_hbm, src_hbm, dst_hbm, g1_hbm, g2_hbm,
               idx_s, idx_d, buf_a, buf_b, sg, st):
    wid = lax.axis_index("s") * _NC + lax.axis_index("c")
    base = wid * _PER_W
    pltpu.sync_copy(src_hbm.at[pl.ds(base, _PER_W)], idx_s)
    pltpu.sync_copy(dst_hbm.at[pl.ds(base, _PER_W)], idx_d)

    def issue_gather(ci, p):
        off = ci * _C
        pltpu.async_copy(t1_hbm.at[idx_s.at[pl.ds(off, _C)]],
                         buf_a.at[p], sg[p])
        pltpu.async_copy(t2_hbm.at[idx_d.at[pl.ds(off, _C)]],
                         buf_b.at[p], sg[p])

    def wait_gather(p):
        pltpu.make_async_copy(t1_hbm.at[idx_s.at[pl.ds(0, _C)]],
                              buf_a.at[p], sg[p]).wait()
        pltpu.make_async_copy(t2_hbm.at[idx_d.at[pl.ds(0, _C)]],
                              buf_b.at[p], sg[p]).wait()

    def issue_store(ci, p):
        off = ci * _C
        pltpu.async_copy(buf_a.at[p], g1_hbm.at[pl.ds(base + off, _C)], st[p])
        pltpu.async_copy(buf_b.at[p], g2_hbm.at[pl.ds(base + off, _C)], st[p])

    def wait_store(p):
        pltpu.make_async_copy(buf_a.at[p],
                              g1_hbm.at[pl.ds(base, _C)], st[p]).wait()
        pltpu.make_async_copy(buf_b.at[p],
                              g2_hbm.at[pl.ds(base, _C)], st[p]).wait()

    issue_gather(0, 0)
    issue_gather(1, 1)

    def quad(t, carry):
        for i in range(_NRING):
            c = 4 * t + i
            q = (i + 2) % _NRING

            @pl.when(c >= 2)
            def _():
                wait_store(q)

            @pl.when(c + 2 <= _NCHUNK - 1)
            def _():
                issue_gather(c + 2, q)

            wait_gather(i)
            issue_store(c, i)
        return carry

    lax.fori_loop(0, _NCHUNK // _NRING, quad, 0)
    wait_gather(0)
    issue_store(_NCHUNK - 1, 0)
    wait_store(2)
    wait_store(3)
    wait_store(0)


_R = 3200


def _unpack_add(g1, g2):
    lo = (jax.lax.bitcast_convert_type(g1 << 16, jnp.float32)
          + jax.lax.bitcast_convert_type(g2 << 16, jnp.float32))
    hi = (jax.lax.bitcast_convert_type(g1 & _HI, jnp.float32)
          + jax.lax.bitcast_convert_type(g2 & _HI, jnp.float32))
    return lo, hi


def _finish_body(g1_ref, g2_ref, ea_ref, w3_ref, b_ref, out_ref):
    lo, hi = _unpack_add(g1_ref[...], g2_ref[...])
    e = (jnp.dot(ea_ref[...].astype(jnp.bfloat16),
                 w3_ref[...].astype(jnp.bfloat16),
                 preferred_element_type=jnp.float32)
         + b_ref[...])
    out_ref[...] = jnp.concatenate([lo + e[:, :HALF], hi + e[:, HALF:]],
                                   axis=1)


def _finish(g1, g2, edge_attr, W3, b2d):
    return pl.pallas_call(
        _finish_body,
        grid=(N_EDGES // _R,),
        in_specs=[
            pl.BlockSpec((_R, HALF), lambda i: (i, 0)),
            pl.BlockSpec((_R, HALF), lambda i: (i, 0)),
            pl.BlockSpec((_R, EDGE_DIM), lambda i: (i, 0)),
            pl.BlockSpec((EDGE_DIM, OUT_DIM), lambda i: (0, 0)),
            pl.BlockSpec((1, OUT_DIM), lambda i: (0, 0)),
        ],
        out_specs=pl.BlockSpec((_R, OUT_DIM), lambda i: (i, 0)),
        out_shape=jax.ShapeDtypeStruct((N_EDGES, OUT_DIM), jnp.float32),
    )(g1, g2, edge_attr, W3, b2d)


def kernel(x, edge_index, edge_attr, W, b):
    src = edge_index[0].astype(jnp.int32)
    dst = edge_index[1].astype(jnp.int32)
    t1, t2 = _make_tables(x, W)
    g1, g2 = _sc_gather(_pack_words(t1), _pack_words(t2), src, dst)
    return _finish(g1, g2, edge_attr, W[2 * NODE_DIM:, :],
                   b.reshape(1, OUT_DIM))


# speedup vs baseline: 1.9113x; 1.9113x over previous
import functools

import jax
import jax.numpy as jnp
from jax import lax
from jax.experimental import pallas as pl
from jax.experimental.pallas import tpu as pltpu
from jax.experimental.pallas import tpu_sc as plsc

N_NODES = 10000
N_EDGES = 320000
NODE_DIM = 128
EDGE_DIM = 16
OUT_DIM = 128

_NC = 2
_NS = 16
_NW = _NC * _NS
_PER_W = N_EDGES // _NW
_C = 80
_NCHUNK = _PER_W // _C


def _tables_body(x_ref, w_ref, t1_ref, t2_ref):
    xv = x_ref[...]
    t1_ref[...] = jnp.dot(xv, w_ref[0:NODE_DIM, :],
                          preferred_element_type=jnp.float32)
    t2_ref[...] = jnp.dot(xv, w_ref[NODE_DIM:2 * NODE_DIM, :],
                          preferred_element_type=jnp.float32)


def _make_tables(x, W):
    return pl.pallas_call(
        _tables_body,
        out_shape=(
            jax.ShapeDtypeStruct((N_NODES, OUT_DIM), jnp.float32),
            jax.ShapeDtypeStruct((N_NODES, OUT_DIM), jnp.float32),
        ),
    )(x, W)


_sc_mesh = plsc.VectorSubcoreMesh(core_axis_name="c", subcore_axis_name="s")


@functools.partial(
    pl.kernel,
    out_type=jax.ShapeDtypeStruct((N_EDGES, OUT_DIM), jnp.float32),
    mesh=_sc_mesh,
    scratch_types=[
        pltpu.VMEM((_PER_W,), jnp.int32),
        pltpu.VMEM((_PER_W,), jnp.int32),
        pltpu.VMEM((2, _C, OUT_DIM), jnp.float32),
        pltpu.VMEM((2, _C, OUT_DIM), jnp.float32),
        pltpu.VMEM((2, _C, OUT_DIM), jnp.float32),
        pltpu.SemaphoreType.DMA,
        pltpu.SemaphoreType.DMA,
        pltpu.SemaphoreType.DMA,
        pltpu.SemaphoreType.DMA,
        pltpu.SemaphoreType.DMA,
        pltpu.SemaphoreType.DMA,
    ],
)
def _sc_gather_sum(t1_hbm, t2_hbm, src_hbm, dst_hbm, out_hbm,
                   idx_s, idx_d, buf_a, buf_b, buf_o,
                   sa0, sa1, sb0, sb1, so0, so1):
    wid = lax.axis_index("s") * _NC + lax.axis_index("c")
    base = wid * _PER_W
    pltpu.sync_copy(src_hbm.at[pl.ds(base, _PER_W)], idx_s)
    pltpu.sync_copy(dst_hbm.at[pl.ds(base, _PER_W)], idx_d)

    sa = (sa0, sa1)
    sb = (sb0, sb1)
    so = (so0, so1)

    def issue(ci, p):
        off = ci * _C
        pltpu.async_copy(t1_hbm.at[idx_s.at[pl.ds(off, _C)]],
                         buf_a.at[p], sa[p])
        pltpu.async_copy(t2_hbm.at[idx_d.at[pl.ds(off, _C)]],
                         buf_b.at[p], sb[p])

    def wait_gather(p):
        pltpu.make_async_copy(t1_hbm.at[idx_s.at[pl.ds(0, _C)]],
                              buf_a.at[p], sa[p]).wait()
        pltpu.make_async_copy(t2_hbm.at[idx_d.at[pl.ds(0, _C)]],
                              buf_b.at[p], sb[p]).wait()

    def wait_store(p):
        pltpu.make_async_copy(buf_o.at[p],
                              out_hbm.at[pl.ds(base, _C)], so[p]).wait()

    def add_store(ci, p):
        def row(r, c2):
            for k in range(OUT_DIM // 16):
                sl = pl.ds(k * 16, 16)
                buf_o[p, r, sl] = buf_a[p, r, sl] + buf_b[p, r, sl]
            return c2

        lax.fori_loop(0, _C, row, 0)
        pltpu.async_copy(buf_o.at[p], out_hbm.at[pl.ds(base + ci * _C, _C)],
                         so[p])

    issue(0, 0)

    def pair(t, carry):
        c0 = 2 * t
        issue(c0 + 1, 1)
        wait_gather(0)

        @pl.when(t >= 1)
        def _():
            wait_store(0)

        add_store(c0, 0)
        issue(c0 + 2, 0)
        wait_gather(1)

        @pl.when(t >= 1)
        def _():
            wait_store(1)

        add_store(c0 + 1, 1)
        return carry

    lax.fori_loop(0, (_NCHUNK - 1) // 2, pair, 0)
    wait_gather(0)
    wait_store(0)
    add_store(_NCHUNK - 1, 0)
    wait_store(1)
    wait_store(0)


_R = 3200


def _finish_body(g_ref, ea_ref, w3_ref, b_ref, out_ref):
    e = lax.dot_general(ea_ref[...], w3_ref[...],
                        dimension_numbers=(((0,), (0,)), ((), ())),
                        preferred_element_type=jnp.float32)
    out_ref[...] = g_ref[...] + e + b_ref[...]


def _finish(g, ea_t, W3, b2d):
    return pl.pallas_call(
        _finish_body,
        grid=(N_EDGES // _R,),
        in_specs=[
            pl.BlockSpec((_R, OUT_DIM), lambda i: (i, 0)),
            pl.BlockSpec((EDGE_DIM, _R), lambda i: (0, i)),
            pl.BlockSpec((EDGE_DIM, OUT_DIM), lambda i: (0, 0)),
            pl.BlockSpec((1, OUT_DIM), lambda i: (0, 0)),
        ],
        out_specs=pl.BlockSpec((_R, OUT_DIM), lambda i: (i, 0)),
        out_shape=jax.ShapeDtypeStruct((N_EDGES, OUT_DIM), jnp.float32),
    )(g, ea_t, W3, b2d)


def kernel(x, edge_index, edge_attr, W, b):
    src = edge_index[0].astype(jnp.int32)
    dst = edge_index[1].astype(jnp.int32)
    t1, t2 = _make_tables(x, W)
    g = _sc_gather_sum(t1, t2, src, dst)
    return _finish(g, edge_attr.T, W[2 * NODE_DIM:, :],
                   b.reshape(1, OUT_DIM))


# speedup vs baseline: 1.9309x; 1.0102x over previous
import functools

import jax
import jax.numpy as jnp
from jax import lax
from jax.experimental import pallas as pl
from jax.experimental.pallas import tpu as pltpu
from jax.experimental.pallas import tpu_sc as plsc

N_NODES = 10000
N_EDGES = 320000
NODE_DIM = 128
EDGE_DIM = 16
OUT_DIM = 128

_NC = 2
_NS = 16
_NW = _NC * _NS
_PER_W = N_EDGES // _NW
_C = 80
_NCHUNK = _PER_W // _C


def _tables_body(x_ref, w_ref, t1_ref, t2_ref):
    xv = x_ref[...]
    t1_ref[...] = jnp.dot(xv, w_ref[0:NODE_DIM, :],
                          preferred_element_type=jnp.float32)
    t2_ref[...] = jnp.dot(xv, w_ref[NODE_DIM:2 * NODE_DIM, :],
                          preferred_element_type=jnp.float32)


def _make_tables(x, W):
    return pl.pallas_call(
        _tables_body,
        out_shape=(
            jax.ShapeDtypeStruct((N_NODES, OUT_DIM), jnp.float32),
            jax.ShapeDtypeStruct((N_NODES, OUT_DIM), jnp.float32),
        ),
    )(x, W)


_sc_mesh = plsc.VectorSubcoreMesh(core_axis_name="c", subcore_axis_name="s")


@functools.partial(
    pl.kernel,
    out_type=jax.ShapeDtypeStruct((N_EDGES, OUT_DIM), jnp.float32),
    mesh=_sc_mesh,
    scratch_types=[
        pltpu.VMEM((_PER_W,), jnp.int32),
        pltpu.VMEM((_PER_W,), jnp.int32),
        pltpu.VMEM((3, _C, OUT_DIM), jnp.float32),
        pltpu.VMEM((3, _C, OUT_DIM), jnp.float32),
        [pltpu.SemaphoreType.DMA] * 3,
        [pltpu.SemaphoreType.DMA] * 3,
    ],
)
def _sc_gather_sum(t1_hbm, t2_hbm, src_hbm, dst_hbm, out_hbm,
                   idx_s, idx_d, buf_a, buf_b, sg, st):
    wid = lax.axis_index("s") * _NC + lax.axis_index("c")
    base = wid * _PER_W
    pltpu.sync_copy(src_hbm.at[pl.ds(base, _PER_W)], idx_s)
    pltpu.sync_copy(dst_hbm.at[pl.ds(base, _PER_W)], idx_d)

    def issue(ci, p):
        off = ci * _C
        pltpu.async_copy(t1_hbm.at[idx_s.at[pl.ds(off, _C)]],
                         buf_a.at[p], sg[p])
        pltpu.async_copy(t2_hbm.at[idx_d.at[pl.ds(off, _C)]],
                         buf_b.at[p], sg[p])

    def wait_gather(p):
        pltpu.make_async_copy(t1_hbm.at[idx_s.at[pl.ds(0, _C)]],
                              buf_a.at[p], sg[p]).wait()
        pltpu.make_async_copy(t2_hbm.at[idx_d.at[pl.ds(0, _C)]],
                              buf_b.at[p], sg[p]).wait()

    def wait_store(p):
        pltpu.make_async_copy(buf_a.at[p],
                              out_hbm.at[pl.ds(base, _C)], st[p]).wait()

    def add_store(ci, p):
        def row(r, c2):
            for k in range(OUT_DIM // 16):
                sl = pl.ds(k * 16, 16)
                plsc.addupdate(buf_a.at[p, r, sl], buf_b[p, r, sl])
            return c2

        lax.fori_loop(0, _C, row, 0)
        pltpu.async_copy(buf_a.at[p], out_hbm.at[pl.ds(base + ci * _C, _C)],
                         st[p])

    issue(0, 0)
    issue(1, 1)

    def trio(t, carry):
        for i in range(3):
            c = 3 * t + i
            q = (i + 2) % 3

            @pl.when(c + 2 <= _NCHUNK - 1)
            def _():
                @pl.when(c >= 1)
                def _():
                    wait_store(q)

                issue(c + 2, q)

            wait_gather(i)
            add_store(c, i)
        return carry

    lax.fori_loop(0, _NCHUNK // 3, trio, 0)
    wait_gather(0)
    add_store(_NCHUNK - 2, 0)
    wait_gather(1)
    add_store(_NCHUNK - 1, 1)
    wait_store(2)
    wait_store(0)
    wait_store(1)


_R = 3200


def _finish_body(g_ref, ea_ref, w3_ref, b_ref, out_ref):
    e = lax.dot_general(ea_ref[...], w3_ref[...],
                        dimension_numbers=(((0,), (0,)), ((), ())),
                        preferred_element_type=jnp.float32)
    out_ref[...] = g_ref[...] + e + b_ref[...]


def _finish(g, ea_t, W3, b2d):
    return pl.pallas_call(
        _finish_body,
        grid=(N_EDGES // _R,),
        in_specs=[
            pl.BlockSpec((_R, OUT_DIM), lambda i: (i, 0)),
            pl.BlockSpec((EDGE_DIM, _R), lambda i: (0, i)),
            pl.BlockSpec((EDGE_DIM, OUT_DIM), lambda i: (0, 0)),
            pl.BlockSpec((1, OUT_DIM), lambda i: (0, 0)),
        ],
        out_specs=pl.BlockSpec((_R, OUT_DIM), lambda i: (i, 0)),
        out_shape=jax.ShapeDtypeStruct((N_EDGES, OUT_DIM), jnp.float32),
    )(g, ea_t, W3, b2d)


def kernel(x, edge_index, edge_attr, W, b):
    src = edge_index[0].astype(jnp.int32)
    dst = edge_index[1].astype(jnp.int32)
    t1, t2 = _make_tables(x, W)
    g = _sc_gather_sum(t1, t2, src, dst)
    return _finish(g, edge_attr.T, W[2 * NODE_DIM:, :],
                   b.reshape(1, OUT_DIM))


# speedup vs baseline: 2.1236x; 1.0998x over previous
import functools

import jax
import jax.numpy as jnp
from jax import lax
from jax.experimental import pallas as pl
from jax.experimental.pallas import tpu as pltpu
from jax.experimental.pallas import tpu_sc as plsc

N_NODES = 10000
N_EDGES = 320000
NODE_DIM = 128
EDGE_DIM = 16
OUT_DIM = 128

_NC = 2
_NS = 16
_NW = _NC * _NS
_PER_W = N_EDGES // _NW
_C = 80
_NCHUNK = _PER_W // _C


def _tables_body(x_ref, w_ref, t1_ref, t2_ref):
    xv = x_ref[...]
    t1_ref[...] = jnp.dot(xv, w_ref[0:NODE_DIM, :],
                          preferred_element_type=jnp.float32)
    t2_ref[...] = jnp.dot(xv, w_ref[NODE_DIM:2 * NODE_DIM, :],
                          preferred_element_type=jnp.float32)


def _make_tables(x, W):
    return pl.pallas_call(
        _tables_body,
        out_shape=(
            jax.ShapeDtypeStruct((N_NODES, OUT_DIM), jnp.float32),
            jax.ShapeDtypeStruct((N_NODES, OUT_DIM), jnp.float32),
        ),
    )(x, W)


_sc_mesh = plsc.VectorSubcoreMesh(core_axis_name="c", subcore_axis_name="s")


@functools.partial(
    pl.kernel,
    out_type=jax.ShapeDtypeStruct((N_EDGES, OUT_DIM), jnp.float32),
    mesh=_sc_mesh,
    scratch_types=[
        pltpu.VMEM((_PER_W,), jnp.int32),
        pltpu.VMEM((_PER_W,), jnp.int32),
        pltpu.VMEM((3, _C, OUT_DIM), jnp.float32),
        pltpu.VMEM((3, _C, OUT_DIM), jnp.float32),
        [pltpu.SemaphoreType.DMA] * 3,
        [pltpu.SemaphoreType.DMA] * 3,
    ],
)
def _sc_gather_sum(t1_hbm, t2_hbm, src_hbm, dst_hbm, out_hbm,
                   idx_s, idx_d, buf_a, buf_b, sg, st):
    wid = lax.axis_index("s") * _NC + lax.axis_index("c")
    base = wid * _PER_W
    pltpu.sync_copy(src_hbm.at[pl.ds(base, _PER_W)], idx_s)
    pltpu.sync_copy(dst_hbm.at[pl.ds(base, _PER_W)], idx_d)

    def issue(ci, p):
        off = ci * _C
        pltpu.async_copy(t1_hbm.at[idx_s.at[pl.ds(off, _C)]],
                         buf_a.at[p], sg[p])
        pltpu.async_copy(t2_hbm.at[idx_d.at[pl.ds(off, _C)]],
                         buf_b.at[p], sg[p])

    def wait_gather(p):
        pltpu.make_async_copy(t1_hbm.at[idx_s.at[pl.ds(0, _C)]],
                              buf_a.at[p], sg[p]).wait()
        pltpu.make_async_copy(t2_hbm.at[idx_d.at[pl.ds(0, _C)]],
                              buf_b.at[p], sg[p]).wait()

    def wait_store(p):
        pltpu.make_async_copy(buf_a.at[p],
                              out_hbm.at[pl.ds(base, _C)], st[p]).wait()

    def add_store(ci, p):
        def row(r, c2):
            for k in range(OUT_DIM // 16):
                sl = pl.ds(k * 16, 16)
                plsc.addupdate(buf_a.at[p, r, sl], buf_b[p, r, sl])
            return c2

        lax.fori_loop(0, _C, row, 0)
        pltpu.async_copy(buf_a.at[p], out_hbm.at[pl.ds(base + ci * _C, _C)],
                         st[p])

    issue(0, 0)
    issue(1, 1)

    def trio(t, carry):
        for i in range(3):
            c = 3 * t + i
            q = (i + 2) % 3

            @pl.when(c + 2 <= _NCHUNK - 1)
            def _():
                @pl.when(c >= 1)
                def _():
                    wait_store(q)

                issue(c + 2, q)

            wait_gather(i)
            add_store(c, i)
        return carry

    lax.fori_loop(0, _NCHUNK // 3, trio, 0)
    wait_gather(0)
    add_store(_NCHUNK - 2, 0)
    wait_gather(1)
    add_store(_NCHUNK - 1, 1)
    wait_store(2)
    wait_store(0)
    wait_store(1)


_R = 6400


def _finish_body(g_ref, ea_ref, w3_ref, b_ref, out_ref):
    e = lax.dot_general(ea_ref[...], w3_ref[...],
                        dimension_numbers=(((0,), (0,)), ((), ())),
                        preferred_element_type=jnp.float32)
    out_ref[...] = g_ref[...] + e + b_ref[...]


def _finish(g, ea_t, W3, b2d):
    return pl.pallas_call(
        _finish_body,
        grid=(N_EDGES // _R,),
        in_specs=[
            pl.BlockSpec((_R, OUT_DIM), lambda i: (i, 0)),
            pl.BlockSpec((EDGE_DIM, _R), lambda i: (0, i)),
            pl.BlockSpec((EDGE_DIM, OUT_DIM), lambda i: (0, 0)),
            pl.BlockSpec((1, OUT_DIM), lambda i: (0, 0)),
        ],
        out_specs=pl.BlockSpec((_R, OUT_DIM), lambda i: (i, 0)),
        out_shape=jax.ShapeDtypeStruct((N_EDGES, OUT_DIM), jnp.float32),
    )(g, ea_t, W3, b2d)


def kernel(x, edge_index, edge_attr, W, b):
    src = edge_index[0].astype(jnp.int32)
    dst = edge_index[1].astype(jnp.int32)
    t1, t2 = _make_tables(x, W)
    g = _sc_gather_sum(t1, t2, src, dst)
    return _finish(g, edge_attr.T, W[2 * NODE_DIM:, :],
                   b.reshape(1, OUT_DIM))


# speedup vs baseline: 2.1647x; 1.0194x over previous
import functools

import jax
import jax.numpy as jnp
from jax import lax
from jax.experimental import pallas as pl
from jax.experimental.pallas import tpu as pltpu
from jax.experimental.pallas import tpu_sc as plsc

N_NODES = 10000
N_EDGES = 320000
NODE_DIM = 128
EDGE_DIM = 16
OUT_DIM = 128

_NC = 2
_NS = 16
_NW = _NC * _NS
_PER_W = N_EDGES // _NW
_C = 80
_NCHUNK = _PER_W // _C


def _tables_body(x_ref, w_ref, t1_ref, t2_ref):
    xv = x_ref[...]
    t1_ref[...] = jnp.dot(xv, w_ref[0:NODE_DIM, :],
                          preferred_element_type=jnp.float32)
    t2_ref[...] = jnp.dot(xv, w_ref[NODE_DIM:2 * NODE_DIM, :],
                          preferred_element_type=jnp.float32)


def _make_tables(x, W):
    return pl.pallas_call(
        _tables_body,
        out_shape=(
            jax.ShapeDtypeStruct((N_NODES, OUT_DIM), jnp.float32),
            jax.ShapeDtypeStruct((N_NODES, OUT_DIM), jnp.float32),
        ),
    )(x, W)


_sc_mesh = plsc.VectorSubcoreMesh(core_axis_name="c", subcore_axis_name="s")


@functools.partial(
    pl.kernel,
    out_type=jax.ShapeDtypeStruct((N_EDGES, OUT_DIM), jnp.float32),
    mesh=_sc_mesh,
    scratch_types=[
        pltpu.VMEM((_PER_W,), jnp.int32),
        pltpu.VMEM((_PER_W,), jnp.int32),
        pltpu.VMEM((3, _C, OUT_DIM), jnp.float32),
        pltpu.VMEM((3, _C, OUT_DIM), jnp.float32),
        [pltpu.SemaphoreType.DMA] * 3,
        [pltpu.SemaphoreType.DMA] * 3,
    ],
)
def _sc_gather_sum(t1_hbm, t2_hbm, src_hbm, dst_hbm, out_hbm,
                   idx_s, idx_d, buf_a, buf_b, sg, st):
    wid = lax.axis_index("s") * _NC + lax.axis_index("c")
    base = wid * _PER_W
    pltpu.sync_copy(src_hbm.at[pl.ds(base, _PER_W)], idx_s)
    pltpu.sync_copy(dst_hbm.at[pl.ds(base, _PER_W)], idx_d)

    def issue(ci, p):
        off = ci * _C
        pltpu.async_copy(t1_hbm.at[idx_s.at[pl.ds(off, _C)]],
                         buf_a.at[p], sg[p])
        pltpu.async_copy(t2_hbm.at[idx_d.at[pl.ds(off, _C)]],
                         buf_b.at[p], sg[p])

    def wait_gather(p):
        pltpu.make_async_copy(t1_hbm.at[idx_s.at[pl.ds(0, _C)]],
                              buf_a.at[p], sg[p]).wait()
        pltpu.make_async_copy(t2_hbm.at[idx_d.at[pl.ds(0, _C)]],
                              buf_b.at[p], sg[p]).wait()

    def wait_store(p):
        pltpu.make_async_copy(buf_a.at[p],
                              out_hbm.at[pl.ds(base, _C)], st[p]).wait()

    def add_store(ci, p):
        def row(r, c2):
            for k in range(OUT_DIM // 16):
                sl = pl.ds(k * 16, 16)
                plsc.addupdate(buf_a.at[p, r, sl], buf_b[p, r, sl])
            return c2

        lax.fori_loop(0, _C, row, 0)
        pltpu.async_copy(buf_a.at[p], out_hbm.at[pl.ds(base + ci * _C, _C)],
                         st[p])

    issue(0, 0)
    issue(1, 1)

    def trio(t, carry):
        for i in range(3):
            c = 3 * t + i
            q = (i + 2) % 3

            @pl.when(c + 2 <= _NCHUNK - 1)
            def _():
                @pl.when(c >= 1)
                def _():
                    wait_store(q)

                issue(c + 2, q)

            wait_gather(i)
            add_store(c, i)
        return carry

    lax.fori_loop(0, _NCHUNK // 3, trio, 0)
    wait_gather(0)
    add_store(_NCHUNK - 2, 0)
    wait_gather(1)
    add_store(_NCHUNK - 1, 1)
    wait_store(2)
    wait_store(0)
    wait_store(1)


_R = 12800


def _finish_body(g_ref, ea_ref, w3_ref, b_ref, out_ref):
    e = lax.dot_general(ea_ref[...], w3_ref[...],
                        dimension_numbers=(((0,), (0,)), ((), ())),
                        preferred_element_type=jnp.float32)
    out_ref[...] = g_ref[...] + e + b_ref[...]


def _finish(g, ea_t, W3, b2d):
    return pl.pallas_call(
        _finish_body,
        grid=(N_EDGES // _R,),
        in_specs=[
            pl.BlockSpec((_R, OUT_DIM), lambda i: (i, 0)),
            pl.BlockSpec((EDGE_DIM, _R), lambda i: (0, i)),
            pl.BlockSpec((EDGE_DIM, OUT_DIM), lambda i: (0, 0)),
            pl.BlockSpec((1, OUT_DIM), lambda i: (0, 0)),
        ],
        out_specs=pl.BlockSpec((_R, OUT_DIM), lambda i: (i, 0)),
        out_shape=jax.ShapeDtypeStruct((N_EDGES, OUT_DIM), jnp.float32),
    )(g, ea_t, W3, b2d)


def kernel(x, edge_index, edge_attr, W, b):
    src = edge_index[0].astype(jnp.int32)
    dst = edge_index[1].astype(jnp.int32)
    t1, t2 = _make_tables(x, W)
    g = _sc_gather_sum(t1, t2, src, dst)
    return _finish(g, edge_attr.T, W[2 * NODE_DIM:, :],
                   b.reshape(1, OUT_DIM))
